# baseline (speedup 1.0000x reference)
import functools

import jax
import jax.numpy as jnp
from jax import lax
from jax.experimental import pallas as pl
from jax.experimental.pallas import tpu as pltpu
from jax.experimental.pallas import tpu_sc as plsc

_NC = 2
_NS = 16
_NW = _NC * _NS

_RB = 64
_EC = 88
_SCAN = 64

_LANES = 16


def _mm_body(a_ref, w_ref, o_ref, *, relu_in, log_softmax_out):
    a = a_ref[...]
    if relu_in:
        a = jnp.maximum(a, 0.0)
    h = jnp.dot(a, w_ref[...], preferred_element_type=jnp.float32)
    if log_softmax_out:
        m = jnp.max(h, axis=1, keepdims=True)
        s = h - m
        h = s - jnp.log(jnp.sum(jnp.exp(s), axis=1, keepdims=True))
    o_ref[...] = h


def _matmul(a, w, *, bm, relu_in=False, log_softmax_out=False):
    m, k = a.shape
    n = w.shape[1]
    return pl.pallas_call(
        functools.partial(_mm_body, relu_in=relu_in,
                          log_softmax_out=log_softmax_out),
        out_shape=jax.ShapeDtypeStruct((m, n), jnp.float32),
        grid=(m // bm,),
        in_specs=[
            pl.BlockSpec((bm, k), lambda i: (i, 0)),
            pl.BlockSpec((k, n), lambda i: (0, 0)),
        ],
        out_specs=pl.BlockSpec((bm, n), lambda i: (i, 0)),
    )(a, w)


def _spmm_sc(h, rp, ci, vals, *, n_rows, n_edges, rpw, rp_win):
    npad, d = h.shape
    nfb = d // _LANES

    def _sload(ref, i):
        return ref[pl.ds(i, _LANES)][0]

    def body(h_hbm, rp_hbm, ci_hbm, vals_hbm, out_hbm,
             rp_v, idx0, idx1, vv0, vv1, eb0, eb1, acc,
             sg0, sg1, scv0, scv1):
        idx = (idx0, idx1)
        vv = (vv0, vv1)
        eb = (eb0, eb1)
        sg = (sg0, sg1)
        scv = (scv0, scv1)
        zero16 = lax.broadcast_in_dim(jnp.float32(0.0), (_LANES,), ())
        c_ax = lax.axis_index("c")
        s_ax = lax.axis_index("s")
        w = s_ax * _NC + c_ax
        r0 = w * rpw
        pltpu.sync_copy(rp_hbm.at[pl.ds(r0, rp_win)], rp_v)
        nsub = rpw // _RB

        def once(pred, fn, carry=0):
            return lax.fori_loop(0, jnp.where(pred, 1, 0),
                                 lambda _i, cy: fn(cy), carry)

        def issue_cv(cbase, b):
            pltpu.async_copy(ci_hbm.at[pl.ds(cbase, _EC)], idx[b], scv[b])
            pltpu.async_copy(vals_hbm.at[pl.ds(cbase, _EC + _LANES)],
                             vv[b], scv[b])

        def wait_cv(cbase, b):
            pltpu.make_async_copy(ci_hbm.at[pl.ds(cbase, _EC)],
                                  idx[b], scv[b]).wait()
            pltpu.make_async_copy(vals_hbm.at[pl.ds(cbase, _EC + _LANES)],
                                  vv[b], scv[b]).wait()

        def sub_block(sb, _):
            rb0 = sb * _RB

            def zrow(r, _2):
                for fb in range(nfb):
                    acc[r, pl.ds(fb * _LANES, _LANES)] = zero16
                return 0
            lax.fori_loop(0, _RB, zrow, 0)

            e_lo = _sload(rp_v, rb0)
            e_hi = _sload(rp_v, rb0 + _RB)
            base0 = e_lo & jnp.int32(~7)
            nchunks = (e_hi - base0 + (_EC - 1)) // _EC

            def cbase(c):
                return pl.multiple_of(base0 + c * _EC, 8)

            def pro0(cy):
                issue_cv(cbase(0), 0)
                return cy
            once(nchunks > 0, pro0)

            def pro1(cy):
                issue_cv(cbase(1), 1)
                return cy
            once(nchunks > 1, pro1)

            def pro2(cy):
                wait_cv(cbase(0), 0)
                pltpu.async_copy(h_hbm.at[idx[0]], eb[0], sg[0])
                return cy
            once(nchunks > 0, pro2)

            def chunk_body(c, b, rcur):
                base = cbase(c)
                estart = jnp.maximum(e_lo, base)
                eend = jnp.minimum(e_hi, base + _EC)
                pltpu.make_async_copy(h_hbm.at[idx[b]], eb[b], sg[b]).wait()

                def prefetch(cy):
                    wait_cv(cbase(c + 1), 1 - b)
                    pltpu.async_copy(h_hbm.at[idx[1 - b]], eb[1 - b],
                                     sg[1 - b])
                    return cy
                once(c + 1 < nchunks, prefetch)

                def bs(_i, lohi):
                    lo_r, hi_r = lohi
                    mid = (lo_r + hi_r + 1) // 2
                    m = _sload(rp_v, mid) < eend
                    return (jnp.where(m, mid, lo_r),
                            jnp.where(m, hi_r, mid - 1))

                nbs = (_RB + 1).bit_length()
                lo_r, _hi_r = lax.fori_loop(0, nbs, bs, (rb0, rb0 + _RB))
                rlast = lo_r + 1

                def row_visit(r, _2):
                    lo = jnp.maximum(_sload(rp_v, r), estart)
                    hi = jnp.minimum(_sload(rp_v, r + 1), eend)
                    ra = r - rb0

                    def process(_cy):
                        accv = tuple(acc[ra, pl.ds(fb * _LANES, _LANES)]
                                     for fb in range(nfb))
                        zidx = lax.broadcast_in_dim(jnp.int32(0),
                                                    (_LANES,), ())

                        def edge(ee, av):
                            j = ee - base
                            v16 = vv[b][pl.ds(j, _LANES)]
                            v = v16.at[zidx].get(mode="promise_in_bounds")
                            return tuple(
                                av[fb] + v * eb[b][j, pl.ds(fb * _LANES,
                                                            _LANES)]
                                for fb in range(nfb))

                        accv = plsc.parallel_loop(
                            lo, hi, step=1, unroll=2, carry=accv)(edge)
                        for fb in range(nfb):
                            acc[ra, pl.ds(fb * _LANES, _LANES)] = accv[fb]
                        return 0

                    once(lo < hi, process)
                    return 0

                lax.fori_loop(rcur, rlast, row_visit, 0)

                def stage_next(cy):
                    issue_cv(cbase(c + 2), b)
                    return cy
                once(c + 2 < nchunks, stage_next)
                return jnp.maximum(rcur, rlast - 1)

            def pair(p, rcur):
                c0 = 2 * p
                rcur = once(c0 < nchunks,
                            lambda rc: chunk_body(c0, 0, rc), rcur)
                rcur = once(c0 + 1 < nchunks,
                            lambda rc: chunk_body(c0 + 1, 1, rc), rcur)
                return rcur

            lax.fori_loop(0, (nchunks + 1) // 2, pair, rb0)
            pltpu.sync_copy(acc, out_hbm.at[pl.ds(r0 + rb0, _RB)])
            return 0

        lax.fori_loop(0, nsub, sub_block, 0)

    mesh = plsc.VectorSubcoreMesh(core_axis_name="c", subcore_axis_name="s",
                                  num_cores=_NC, num_subcores=_NS)
    f = pl.kernel(
        body,
        out_type=jax.ShapeDtypeStruct((npad, d), jnp.float32),
        mesh=mesh,
        scratch_types=[
            pltpu.VMEM((rp_win,), jnp.int32),
            pltpu.VMEM((_EC,), jnp.int32),
            pltpu.VMEM((_EC,), jnp.int32),
            pltpu.VMEM((_EC + _LANES,), jnp.float32),
            pltpu.VMEM((_EC + _LANES,), jnp.float32),
            pltpu.VMEM((_EC, d), jnp.float32),
            pltpu.VMEM((_EC, d), jnp.float32),
            pltpu.VMEM((_RB, d), jnp.float32),
            pltpu.SemaphoreType.DMA,
            pltpu.SemaphoreType.DMA,
            pltpu.SemaphoreType.DMA,
            pltpu.SemaphoreType.DMA,
        ],
    )
    return f(h, rp, ci, vals)


def kernel(x, rowPtr, colInd, values, W1, W2, W3):
    n, f_in = x.shape
    e = values.shape[0]
    h_dim = W1.shape[1]

    bm = 1024
    npad = ((n + bm - 1) // bm) * bm

    rpw = (((n + _NW - 1) // _NW + _RB - 1) // _RB) * _RB
    nsub = rpw // _RB
    rp_win = (nsub - 1) * _RB + max(_SCAN, _RB + 1) + _LANES
    rp_win = ((rp_win + 7) // 8) * 8
    rp_pad = (_NW - 1) * rpw + rp_win
    assert _NW * rpw <= npad and rpw % _RB == 0

    rp = rowPtr.at[0].set(0).at[n].set(e)
    rp = jnp.pad(rp, (0, rp_pad - (n + 1)), constant_values=e)
    ci = jnp.pad(colInd, (0, _EC))
    vals = jnp.pad(values, (0, _EC + _LANES))
    xp = jnp.pad(x, ((0, npad - n), (0, 0)))

    spmm = functools.partial(_spmm_sc, n_rows=n, n_edges=e, rpw=rpw,
                             rp_win=rp_win)

    h1 = _matmul(xp, W1, bm=bm)
    a1 = spmm(h1, rp, ci, vals)
    h2 = _matmul(a1, W2, bm=bm, relu_in=True)
    a2 = spmm(h2, rp, ci, vals)
    out = _matmul(a2, W3, bm=bm, relu_in=True, log_softmax_out=True)
    return out[:n]


# speedup vs baseline: 1.0140x; 1.0140x over previous
import functools

import jax
import jax.numpy as jnp
from jax import lax
from jax.experimental import pallas as pl
from jax.experimental.pallas import tpu as pltpu
from jax.experimental.pallas import tpu_sc as plsc

_NC = 2
_NS = 16
_NW = _NC * _NS

_RB = 64
_EC = 88
_SCAN = 64

_LANES = 16


def _mm_body(a_ref, w_ref, o_ref, *, relu_in, log_softmax_out):
    a = a_ref[...]
    if relu_in:
        a = jnp.maximum(a, 0.0)
    h = jnp.dot(a, w_ref[...], preferred_element_type=jnp.float32)
    if log_softmax_out:
        m = jnp.max(h, axis=1, keepdims=True)
        s = h - m
        h = s - jnp.log(jnp.sum(jnp.exp(s), axis=1, keepdims=True))
    o_ref[...] = h


def _matmul(a, w, *, bm, relu_in=False, log_softmax_out=False):
    m, k = a.shape
    n = w.shape[1]
    return pl.pallas_call(
        functools.partial(_mm_body, relu_in=relu_in,
                          log_softmax_out=log_softmax_out),
        out_shape=jax.ShapeDtypeStruct((m, n), jnp.float32),
        grid=(m // bm,),
        in_specs=[
            pl.BlockSpec((bm, k), lambda i: (i, 0)),
            pl.BlockSpec((k, n), lambda i: (0, 0)),
        ],
        out_specs=pl.BlockSpec((bm, n), lambda i: (i, 0)),
    )(a, w)


def _spmm_sc(h, rp, ci, vals, *, n_rows, n_edges, rpw, rp_win):
    npad, d = h.shape
    nfb = d // _LANES

    def _sload(ref, i):
        return ref[pl.ds(i, _LANES)][0]

    def body(h_hbm, rp_hbm, ci_hbm, vals_hbm, out_hbm,
             rp_v, idx0, idx1, vv0, vv1, eb0, eb1, acc,
             sg0, sg1, scv0, scv1):
        idx = (idx0, idx1)
        vv = (vv0, vv1)
        eb = (eb0, eb1)
        sg = (sg0, sg1)
        scv = (scv0, scv1)
        zero16 = lax.broadcast_in_dim(jnp.float32(0.0), (_LANES,), ())
        c_ax = lax.axis_index("c")
        s_ax = lax.axis_index("s")
        w = s_ax * _NC + c_ax
        r0 = w * rpw
        pltpu.sync_copy(rp_hbm.at[pl.ds(r0, rp_win)], rp_v)
        nsub = rpw // _RB

        def once(pred, fn, carry=0):
            return lax.fori_loop(0, jnp.where(pred, 1, 0),
                                 lambda _i, cy: fn(cy), carry)

        def issue_cv(cbase, b):
            pltpu.async_copy(ci_hbm.at[pl.ds(cbase, _EC)], idx[b], scv[b])
            pltpu.async_copy(vals_hbm.at[pl.ds(cbase, _EC + _LANES)],
                             vv[b], scv[b])

        def wait_cv(cbase, b):
            pltpu.make_async_copy(ci_hbm.at[pl.ds(cbase, _EC)],
                                  idx[b], scv[b]).wait()
            pltpu.make_async_copy(vals_hbm.at[pl.ds(cbase, _EC + _LANES)],
                                  vv[b], scv[b]).wait()

        def sub_block(sb, _):
            rb0 = sb * _RB

            def zrow(r, _2):
                for fb in range(nfb):
                    acc[r, pl.ds(fb * _LANES, _LANES)] = zero16
                return 0
            lax.fori_loop(0, _RB, zrow, 0)

            e_lo = _sload(rp_v, rb0)
            e_hi = _sload(rp_v, rb0 + _RB)
            base0 = e_lo & jnp.int32(~7)
            nchunks = (e_hi - base0 + (_EC - 1)) // _EC

            def cbase(c):
                return pl.multiple_of(base0 + c * _EC, 8)

            def pro0(cy):
                issue_cv(cbase(0), 0)
                return cy
            once(nchunks > 0, pro0)

            def pro1(cy):
                issue_cv(cbase(1), 1)
                return cy
            once(nchunks > 1, pro1)

            def pro2(cy):
                wait_cv(cbase(0), 0)
                pltpu.async_copy(h_hbm.at[idx[0]], eb[0], sg[0])
                return cy
            once(nchunks > 0, pro2)

            def chunk_body(c, b, rcur):
                base = cbase(c)
                estart = jnp.maximum(e_lo, base)
                eend = jnp.minimum(e_hi, base + _EC)
                pltpu.make_async_copy(h_hbm.at[idx[b]], eb[b], sg[b]).wait()

                def prefetch(cy):
                    wait_cv(cbase(c + 1), 1 - b)
                    pltpu.async_copy(h_hbm.at[idx[1 - b]], eb[1 - b],
                                     sg[1 - b])
                    return cy
                once(c + 1 < nchunks, prefetch)

                def bs(_i, lohi):
                    lo_r, hi_r = lohi
                    mid = (lo_r + hi_r + 1) // 2
                    m = _sload(rp_v, mid) < eend
                    return (jnp.where(m, mid, lo_r),
                            jnp.where(m, hi_r, mid - 1))

                nbs = (_RB + 1).bit_length()
                lo_r, _hi_r = lax.fori_loop(0, nbs, bs, (rb0, rb0 + _RB))
                rlast = lo_r + 1

                def row_visit(r, _2):
                    lo = jnp.maximum(_sload(rp_v, r), estart)
                    hi = jnp.minimum(_sload(rp_v, r + 1), eend)
                    ra = r - rb0

                    def process(_cy):
                        accv = tuple(acc[ra, pl.ds(fb * _LANES, _LANES)]
                                     for fb in range(nfb))
                        zidx = lax.broadcast_in_dim(jnp.int32(0),
                                                    (_LANES,), ())

                        def edge(ee, av):
                            j = ee - base
                            v16 = vv[b][pl.ds(j, _LANES)]
                            v = v16.at[zidx].get(mode="promise_in_bounds")
                            return tuple(
                                av[fb] + v * eb[b][j, pl.ds(fb * _LANES,
                                                            _LANES)]
                                for fb in range(nfb))

                        accv = plsc.parallel_loop(
                            lo, hi, step=1, unroll=2, carry=accv)(edge)
                        for fb in range(nfb):
                            acc[ra, pl.ds(fb * _LANES, _LANES)] = accv[fb]
                        return 0

                    once(lo < hi, process)
                    return 0

                lax.fori_loop(rcur, rlast, row_visit, 0)

                def stage_next(cy):
                    issue_cv(cbase(c + 2), b)
                    return cy
                once(c + 2 < nchunks, stage_next)
                return jnp.maximum(rcur, rlast - 1)

            def pair(p, rcur):
                c0 = 2 * p
                rcur = once(c0 < nchunks,
                            lambda rc: chunk_body(c0, 0, rc), rcur)
                rcur = once(c0 + 1 < nchunks,
                            lambda rc: chunk_body(c0 + 1, 1, rc), rcur)
                return rcur

            lax.fori_loop(0, (nchunks + 1) // 2, pair, rb0)
            pltpu.sync_copy(acc, out_hbm.at[pl.ds(r0 + rb0, _RB)])
            return 0

        lax.fori_loop(0, nsub, sub_block, 0)

    mesh = plsc.VectorSubcoreMesh(core_axis_name="c", subcore_axis_name="s",
                                  num_cores=_NC, num_subcores=_NS)
    f = pl.kernel(
        body,
        out_type=jax.ShapeDtypeStruct((npad, d), jnp.float32),
        mesh=mesh,
        scratch_types=[
            pltpu.VMEM((rp_win,), jnp.int32),
            pltpu.VMEM((_EC,), jnp.int32),
            pltpu.VMEM((_EC,), jnp.int32),
            pltpu.VMEM((_EC + _LANES,), jnp.float32),
            pltpu.VMEM((_EC + _LANES,), jnp.float32),
            pltpu.VMEM((_EC, d), jnp.float32),
            pltpu.VMEM((_EC, d), jnp.float32),
            pltpu.VMEM((_RB, d), jnp.float32),
            pltpu.SemaphoreType.DMA,
            pltpu.SemaphoreType.DMA,
            pltpu.SemaphoreType.DMA,
            pltpu.SemaphoreType.DMA,
        ],
    )
    return f(h, rp, ci, vals)


def kernel(x, rowPtr, colInd, values, W1, W2, W3):
    n, f_in = x.shape
    e = values.shape[0]
    h_dim = W1.shape[1]

    bm = 2048
    npad = ((n + bm - 1) // bm) * bm

    rpw = (((n + _NW - 1) // _NW + _RB - 1) // _RB) * _RB
    nsub = rpw // _RB
    rp_win = (nsub - 1) * _RB + max(_SCAN, _RB + 1) + _LANES
    rp_win = ((rp_win + 7) // 8) * 8
    rp_pad = (_NW - 1) * rpw + rp_win
    assert _NW * rpw <= npad and rpw % _RB == 0

    rp = rowPtr.at[0].set(0).at[n].set(e)
    rp = jnp.pad(rp, (0, rp_pad - (n + 1)), constant_values=e)
    ci = jnp.pad(colInd, (0, _EC))
    vals = jnp.pad(values, (0, _EC + _LANES))
    xp = jnp.pad(x, ((0, npad - n), (0, 0)))

    spmm = functools.partial(_spmm_sc, n_rows=n, n_edges=e, rpw=rpw,
                             rp_win=rp_win)

    h1 = _matmul(xp, W1, bm=bm)
    a1 = spmm(h1, rp, ci, vals)
    h2 = _matmul(a1, W2, bm=bm, relu_in=True)
    a2 = spmm(h2, rp, ci, vals)
    out = _matmul(a2, W3, bm=bm, relu_in=True, log_softmax_out=True)
    return out[:n]
